# baseline (speedup 1.0000x reference)
import functools

import jax
import jax.numpy as jnp
from jax import lax
from jax.experimental import pallas as pl
from jax.experimental.pallas import tpu as pltpu
from jax.experimental.pallas import tpu_sc as plsc


def _sc_gather(table, idx):
    V, D = table.shape
    B = idx.shape[0]
    info = plsc.get_sparse_core_info()
    NC, NS = info.num_cores, info.num_subcores
    NW = NC * NS
    assert B % NW == 0
    b_per_w = B // NW
    mesh = plsc.VectorSubcoreMesh(core_axis_name="c", subcore_axis_name="s")

    @functools.partial(
        pl.kernel,
        mesh=mesh,
        out_type=jax.ShapeDtypeStruct((B, D), jnp.float32),
        scratch_types=[
            pltpu.VMEM((b_per_w,), jnp.int32),
            pltpu.VMEM((b_per_w, D), jnp.float32),
            pltpu.SemaphoreType.DMA,
        ],
        compiler_params=pltpu.CompilerParams(use_tc_tiling_on_sc=False),
    )
    def k(table_hbm, idx_hbm, out_hbm, idx_v, rows_v, sem):
        wid = lax.axis_index("s") * NC + lax.axis_index("c")
        base = wid * b_per_w
        pltpu.sync_copy(idx_hbm.at[pl.ds(base, b_per_w)], idx_v)
        pltpu.async_copy(table_hbm.at[idx_v], rows_v, sem).wait()
        pltpu.sync_copy(rows_v, out_hbm.at[pl.ds(base, b_per_w)])

    return k(table, idx)


_VBLK = 2048
_NFULL = 48
_TAIL = 1696
_TAIL_OFF = _NFULL * _VBLK
_NBLK = _NFULL + 1
_RSTR = 32


def _dot_bf(fbf, wv):
    return lax.dot_general(
        fbf,
        wv.astype(jnp.bfloat16),
        dimension_numbers=(((1,), (1,)), ((), ())),
        preferred_element_type=jnp.float32,
    )


def _pass1(flat, W):
    B, K = flat.shape

    def body(flat_hbm, w_hbm, z_hbm,
             flat_v, flat_bf, w0, w1, wt, z0, z1,
             sem_f, sem_r, sem_w, sem_t):
        wbufs = (w0, w1)
        zbufs = (z0, z1)

        def start_read(i):
            pltpu.make_async_copy(
                w_hbm.at[pl.ds(i * _VBLK, _VBLK), :], wbufs[i % 2],
                sem_r.at[i % 2],
            ).start()

        def wait_read(i):
            pltpu.make_async_copy(
                w_hbm.at[pl.ds(0, _VBLK), :], wbufs[i % 2], sem_r.at[i % 2]
            ).wait()

        def start_write(i):
            pltpu.make_async_copy(
                zbufs[i % 2], z_hbm.at[i], sem_w.at[i % 2]
            ).start()

        def wait_write(i):
            pltpu.make_async_copy(
                zbufs[i % 2], z_hbm.at[0], sem_w.at[i % 2]
            ).wait()

        pltpu.make_async_copy(flat_hbm, flat_v, sem_f).start()
        start_read(0)
        start_read(1)
        pltpu.make_async_copy(
            w_hbm.at[pl.ds(_TAIL_OFF, _TAIL), :], wt, sem_t
        ).start()
        pltpu.make_async_copy(flat_hbm, flat_v, sem_f).wait()
        flat_bf[...] = flat_v[...].astype(jnp.bfloat16)

        for i in range(_NFULL):
            wait_read(i)
            if i >= 2:
                wait_write(i - 2)
            zbufs[i % 2][...] = _dot_bf(flat_bf[...], wbufs[i % 2][...])
            start_write(i)
            if i + 2 < _NFULL:
                start_read(i + 2)

        pltpu.make_async_copy(
            w_hbm.at[pl.ds(0, _TAIL), :], wt, sem_t
        ).wait()
        wait_write(_NFULL - 2)
        z0[:, : _TAIL] = _dot_bf(flat_bf[...], wt[...])
        pltpu.make_async_copy(z0, z_hbm.at[_NFULL], sem_t).start()
        wait_write(_NFULL - 1)
        pltpu.make_async_copy(z0, z_hbm.at[0], sem_t).wait()

    return pl.pallas_call(
        body,
        in_specs=[
            pl.BlockSpec(memory_space=pl.ANY),
            pl.BlockSpec(memory_space=pl.ANY),
        ],
        out_specs=pl.BlockSpec(memory_space=pl.ANY),
        out_shape=jax.ShapeDtypeStruct((_NBLK, B, _VBLK), jnp.float32),
        scratch_shapes=[
            pltpu.VMEM((B, K), jnp.float32),
            pltpu.VMEM((B, K), jnp.bfloat16),
            pltpu.VMEM((_VBLK, K), jnp.float32),
            pltpu.VMEM((_VBLK, K), jnp.float32),
            pltpu.VMEM((_TAIL, K), jnp.float32),
            pltpu.VMEM((B, _VBLK), jnp.float32),
            pltpu.VMEM((B, _VBLK), jnp.float32),
            pltpu.SemaphoreType.DMA,
            pltpu.SemaphoreType.DMA((2,)),
            pltpu.SemaphoreType.DMA((2,)),
            pltpu.SemaphoreType.DMA,
        ],
        compiler_params=pltpu.CompilerParams(
            vmem_limit_bytes=64 * 1024 * 1024,
        ),
    )(flat, W)


def _pass2(Z, b2d):
    _, B, _ = Z.shape
    V = b2d.shape[1]
    nstr = B // _RSTR

    def body(z_hbm, b_hbm, out_hbm, b_v, zs0, zs1, st0, st1,
             sem_b, sem_r, sem_w):
        zbufs = (zs0, zs1)
        sbufs = (st0, st1)

        def start_read_dyn(u, s):
            off = jnp.minimum(s, _NBLK - 1)
            pltpu.make_async_copy(
                z_hbm.at[off], zbufs[u], sem_r.at[u]
            ).start()

        def start_read(s):
            start_read_dyn(s % 2, s)

        def wait_read_dyn(u):
            pltpu.make_async_copy(
                z_hbm.at[0], zbufs[u], sem_r.at[u]
            ).wait()

        def wait_read(s):
            wait_read_dyn(s % 2)

        def start_write_dyn(u, s):
            off = pl.multiple_of(s * _RSTR, _RSTR)
            pltpu.make_async_copy(
                sbufs[u], out_hbm.at[pl.ds(off, _RSTR), :], sem_w.at[u]
            ).start()

        def start_write(u, s=None):
            start_write_dyn(u % 2, u if s is None else s)

        def wait_write_dyn(u):
            pltpu.make_async_copy(
                sbufs[u], out_hbm.at[pl.ds(0, _RSTR), :], sem_w.at[u]
            ).wait()

        def wait_write(u):
            wait_write_dyn(u % 2)

        def assemble(u):
            return
            zs = zbufs[u]
            st = sbufs[u]
            for j in range(_NFULL):
                c0 = j * _VBLK
                st[:, c0:c0 + _VBLK] = zs[j] + b_v[:, c0:c0 + _VBLK]
            st[:, _TAIL_OFF:_TAIL_OFF + _TAIL] = (
                zs[_NFULL, :, : _TAIL]
                + b_v[:, _TAIL_OFF:_TAIL_OFF + _TAIL]
            )

        pltpu.make_async_copy(b_hbm, b_v, sem_b).start()
        start_read(0)
        start_read(1)
        pltpu.make_async_copy(b_hbm, b_v, sem_b).wait()

        for u in range(2):
            wait_read(u)
            assemble(u)
            start_write_dyn(u, u)
        for s in range(2, nstr):
            u = s % 2
            start_read_dyn(u, s)
            wait_read_dyn(u)
            wait_write_dyn(u)
            assemble(u)
            start_write_dyn(u, s)

        wait_write(0)
        wait_write(1)

    return pl.pallas_call(
        body,
        in_specs=[
            pl.BlockSpec(memory_space=pl.ANY),
            pl.BlockSpec(memory_space=pl.ANY),
        ],
        out_specs=pl.BlockSpec(memory_space=pl.ANY),
        out_shape=jax.ShapeDtypeStruct((B, V), jnp.float32),
        scratch_shapes=[
            pltpu.VMEM((1, V), jnp.float32),
            pltpu.VMEM((1024, _VBLK), jnp.float32),
            pltpu.VMEM((1024, _VBLK), jnp.float32),
            pltpu.VMEM((_RSTR, V), jnp.float32),
            pltpu.VMEM((_RSTR, V), jnp.float32),
            pltpu.SemaphoreType.DMA,
            pltpu.SemaphoreType.DMA((2,)),
            pltpu.SemaphoreType.DMA((2,)),
        ],
        compiler_params=pltpu.CompilerParams(
            vmem_limit_bytes=64 * 1024 * 1024,
        ),
    )(Z, b2d)


def kernel(inputs, emb_table, W, b):
    api_seq = inputs[0]
    B, N = api_seq.shape
    D = emb_table.shape[1]
    idx = api_seq.reshape(B * N)
    rows = _sc_gather(emb_table, idx)
    flat = rows.reshape(B, N * D)
    Z = _pass1(flat, W)
    return _pass2(Z, b.reshape(1, -1))


# speedup vs baseline: 1.0457x; 1.0457x over previous
import functools

import jax
import jax.numpy as jnp
from jax import lax
from jax.experimental import pallas as pl
from jax.experimental.pallas import tpu as pltpu
from jax.experimental.pallas import tpu_sc as plsc


def _sc_gather(table, idx):
    V, D = table.shape
    B = idx.shape[0]
    info = plsc.get_sparse_core_info()
    NC, NS = info.num_cores, info.num_subcores
    NW = NC * NS
    assert B % NW == 0
    b_per_w = B // NW
    mesh = plsc.VectorSubcoreMesh(core_axis_name="c", subcore_axis_name="s")

    @functools.partial(
        pl.kernel,
        mesh=mesh,
        out_type=jax.ShapeDtypeStruct((B, D), jnp.float32),
        scratch_types=[
            pltpu.VMEM((b_per_w,), jnp.int32),
            pltpu.VMEM((b_per_w, D), jnp.float32),
            pltpu.SemaphoreType.DMA,
        ],
        compiler_params=pltpu.CompilerParams(use_tc_tiling_on_sc=False),
    )
    def k(table_hbm, idx_hbm, out_hbm, idx_v, rows_v, sem):
        wid = lax.axis_index("s") * NC + lax.axis_index("c")
        base = wid * b_per_w
        pltpu.sync_copy(idx_hbm.at[pl.ds(base, b_per_w)], idx_v)
        pltpu.async_copy(table_hbm.at[idx_v], rows_v, sem).wait()
        pltpu.sync_copy(rows_v, out_hbm.at[pl.ds(base, b_per_w)])

    return k(table, idx)


_VBLK = 2048
_NFULL = 48
_TAIL = 1696
_TAIL_OFF = _NFULL * _VBLK
_NBLK = _NFULL + 1
_RSTR = 32


def _dot_bf(fbf, wv):
    return lax.dot_general(
        fbf,
        wv.astype(jnp.bfloat16),
        dimension_numbers=(((1,), (1,)), ((), ())),
        preferred_element_type=jnp.float32,
    )


def _pass1(flat, W):
    B, K = flat.shape

    def body(flat_hbm, w_hbm, z_hbm,
             flat_v, flat_bf, w0, w1, wt, z0, z1,
             sem_f, sem_r, sem_w, sem_t):
        wbufs = (w0, w1)
        zbufs = (z0, z1)

        def start_read(i):
            pltpu.make_async_copy(
                w_hbm.at[pl.ds(i * _VBLK, _VBLK), :], wbufs[i % 2],
                sem_r.at[i % 2],
            ).start()

        def wait_read(i):
            pltpu.make_async_copy(
                w_hbm.at[pl.ds(0, _VBLK), :], wbufs[i % 2], sem_r.at[i % 2]
            ).wait()

        def start_write(i):
            pltpu.make_async_copy(
                zbufs[i % 2], z_hbm.at[i], sem_w.at[i % 2]
            ).start()

        def wait_write(i):
            pltpu.make_async_copy(
                zbufs[i % 2], z_hbm.at[0], sem_w.at[i % 2]
            ).wait()

        pltpu.make_async_copy(flat_hbm, flat_v, sem_f).start()
        start_read(0)
        start_read(1)
        pltpu.make_async_copy(
            w_hbm.at[pl.ds(_TAIL_OFF, _TAIL), :], wt, sem_t
        ).start()
        pltpu.make_async_copy(flat_hbm, flat_v, sem_f).wait()
        flat_bf[...] = flat_v[...].astype(jnp.bfloat16)

        for i in range(_NFULL):
            wait_read(i)
            if i >= 2:
                wait_write(i - 2)
            zbufs[i % 2][...] = _dot_bf(flat_bf[...], wbufs[i % 2][...])
            start_write(i)
            if i + 2 < _NFULL:
                start_read(i + 2)

        pltpu.make_async_copy(
            w_hbm.at[pl.ds(0, _TAIL), :], wt, sem_t
        ).wait()
        wait_write(_NFULL - 2)
        z0[:, : _TAIL] = _dot_bf(flat_bf[...], wt[...])
        pltpu.make_async_copy(z0, z_hbm.at[_NFULL], sem_t).start()
        wait_write(_NFULL - 1)
        pltpu.make_async_copy(z0, z_hbm.at[0], sem_t).wait()

    return pl.pallas_call(
        body,
        in_specs=[
            pl.BlockSpec(memory_space=pl.ANY),
            pl.BlockSpec(memory_space=pl.ANY),
        ],
        out_specs=pl.BlockSpec(memory_space=pl.ANY),
        out_shape=jax.ShapeDtypeStruct((_NBLK, B, _VBLK), jnp.float32),
        scratch_shapes=[
            pltpu.VMEM((B, K), jnp.float32),
            pltpu.VMEM((B, K), jnp.bfloat16),
            pltpu.VMEM((_VBLK, K), jnp.float32),
            pltpu.VMEM((_VBLK, K), jnp.float32),
            pltpu.VMEM((_TAIL, K), jnp.float32),
            pltpu.VMEM((B, _VBLK), jnp.float32),
            pltpu.VMEM((B, _VBLK), jnp.float32),
            pltpu.SemaphoreType.DMA,
            pltpu.SemaphoreType.DMA((2,)),
            pltpu.SemaphoreType.DMA((2,)),
            pltpu.SemaphoreType.DMA,
        ],
        compiler_params=pltpu.CompilerParams(
            vmem_limit_bytes=64 * 1024 * 1024,
        ),
    )(flat, W)


def _pass2(Z, b2d):
    _, B, _ = Z.shape
    V = b2d.shape[1]
    nstr = B // _RSTR

    def body(z_hbm, b_hbm, out_hbm, b_v, zs0, zs1, st0, st1,
             sem_b, sem_r, sem_w):
        zbufs = (zs0, zs1)
        sbufs = (st0, st1)

        def start_read_dyn(u, s):
            off = jnp.minimum(s, _NBLK - 1)
            pltpu.make_async_copy(
                z_hbm.at[off], zbufs[u], sem_r.at[u]
            ).start()

        def start_read(s):
            start_read_dyn(s % 2, s)

        def wait_read_dyn(u):
            pltpu.make_async_copy(
                z_hbm.at[0], zbufs[u], sem_r.at[u]
            ).wait()

        def wait_read(s):
            wait_read_dyn(s % 2)

        def start_write_dyn(u, s):
            off = pl.multiple_of(s * _RSTR, _RSTR)
            pltpu.make_async_copy(
                sbufs[u], out_hbm.at[pl.ds(off, _RSTR), :], sem_w.at[u]
            ).start()

        def start_write(u, s=None):
            start_write_dyn(u % 2, u if s is None else s)

        def wait_write_dyn(u):
            pltpu.make_async_copy(
                sbufs[u], out_hbm.at[pl.ds(0, _RSTR), :], sem_w.at[u]
            ).wait()

        def wait_write(u):
            wait_write_dyn(u % 2)

        def assemble(u):
            return
            zs = zbufs[u]
            st = sbufs[u]
            for j in range(_NFULL):
                c0 = j * _VBLK
                st[:, c0:c0 + _VBLK] = zs[j] + b_v[:, c0:c0 + _VBLK]
            st[:, _TAIL_OFF:_TAIL_OFF + _TAIL] = (
                zs[_NFULL, :, : _TAIL]
                + b_v[:, _TAIL_OFF:_TAIL_OFF + _TAIL]
            )

        pltpu.make_async_copy(b_hbm, b_v, sem_b).start()
        start_read(0)
        start_read(1)
        pltpu.make_async_copy(b_hbm, b_v, sem_b).wait()

        for u in range(2):
            wait_read(u)
            assemble(u)
            start_write_dyn(u, u)
        for s in range(2, nstr):
            u = s % 2
            start_read_dyn(u, s)
            wait_read_dyn(u)
            wait_write_dyn(u)
            assemble(u)
            start_write_dyn(u, s)

        wait_write(0)
        wait_write(1)

    return pl.pallas_call(
        body,
        in_specs=[
            pl.BlockSpec(memory_space=pl.ANY),
            pl.BlockSpec(memory_space=pl.ANY),
        ],
        out_specs=pl.BlockSpec(memory_space=pl.ANY),
        out_shape=jax.ShapeDtypeStruct((B, V), jnp.float32),
        scratch_shapes=[
            pltpu.VMEM((1, V), jnp.float32),
            pltpu.VMEM((1024, _VBLK), jnp.float32),
            pltpu.VMEM((1024, _VBLK), jnp.float32),
            pltpu.VMEM((_RSTR, V), jnp.float32),
            pltpu.VMEM((_RSTR, V), jnp.float32),
            pltpu.SemaphoreType.DMA,
            pltpu.SemaphoreType.DMA((2,)),
            pltpu.SemaphoreType.DMA((2,)),
        ],
        compiler_params=pltpu.CompilerParams(
            vmem_limit_bytes=64 * 1024 * 1024,
        ),
    )(Z, b2d)


def kernel(inputs, emb_table, W, b):
    api_seq = inputs[0]
    B, N = api_seq.shape
    D = emb_table.shape[1]
    idx = api_seq.reshape(B * N)
    rows = _sc_gather(emb_table, idx)
    flat = rows.reshape(B, N * D)
    Z = _pass1(flat, W)
    out = Z.transpose(1, 0, 2).reshape(B, _NBLK * _VBLK)[:, : W.shape[0]]
    return out + b[None, :]


# speedup vs baseline: 1.3481x; 1.2892x over previous
import functools

import jax
import jax.numpy as jnp
from jax import lax
from jax.experimental import pallas as pl
from jax.experimental.pallas import tpu as pltpu
from jax.experimental.pallas import tpu_sc as plsc


def _sc_gather(table, idx):
    V, D = table.shape
    B = idx.shape[0]
    info = plsc.get_sparse_core_info()
    NC, NS = info.num_cores, info.num_subcores
    NW = NC * NS
    assert B % NW == 0
    b_per_w = B // NW
    mesh = plsc.VectorSubcoreMesh(core_axis_name="c", subcore_axis_name="s")

    @functools.partial(
        pl.kernel,
        mesh=mesh,
        out_type=jax.ShapeDtypeStruct((B, D), jnp.float32),
        scratch_types=[
            pltpu.VMEM((b_per_w,), jnp.int32),
            pltpu.VMEM((b_per_w, D), jnp.float32),
            pltpu.SemaphoreType.DMA,
        ],
        compiler_params=pltpu.CompilerParams(use_tc_tiling_on_sc=False),
    )
    def k(table_hbm, idx_hbm, out_hbm, idx_v, rows_v, sem):
        wid = lax.axis_index("s") * NC + lax.axis_index("c")
        base = wid * b_per_w
        pltpu.sync_copy(idx_hbm.at[pl.ds(base, b_per_w)], idx_v)
        pltpu.async_copy(table_hbm.at[idx_v], rows_v, sem).wait()
        pltpu.sync_copy(rows_v, out_hbm.at[pl.ds(base, b_per_w)])

    return k(table, idx)


def _proj_body(flat_ref, w_ref, b_ref, out_ref):
    out_ref[...] = (
        lax.dot_general(
            flat_ref[...].astype(jnp.bfloat16),
            w_ref[...].astype(jnp.bfloat16),
            dimension_numbers=(((1,), (1,)), ((), ())),
            preferred_element_type=jnp.float32,
        )
        + b_ref[...]
    )


def _projection(flat, W, b2d, vblk):
    B, K = flat.shape
    V = W.shape[0]
    nblk = (V + vblk - 1) // vblk
    return pl.pallas_call(
        _proj_body,
        grid=(nblk,),
        in_specs=[
            pl.BlockSpec((B, K), lambda j: (0, 0)),
            pl.BlockSpec((vblk, K), lambda j: (j, 0)),
            pl.BlockSpec((1, vblk), lambda j: (0, j)),
        ],
        out_specs=pl.BlockSpec((B, vblk), lambda j: (0, j)),
        out_shape=jax.ShapeDtypeStruct((B, V), jnp.float32),
    )(flat, W, b2d)


def kernel(inputs, emb_table, W, b):
    api_seq = inputs[0]
    B, N = api_seq.shape
    D = emb_table.shape[1]
    idx = api_seq.reshape(B * N)
    rows = _sc_gather(emb_table, idx)
    flat = rows.reshape(B, N * D)
    return _projection(flat, W, b.reshape(1, -1), vblk=2048)


# speedup vs baseline: 1.3669x; 1.0139x over previous
import functools

import jax
import jax.numpy as jnp
from jax import lax
from jax.experimental import pallas as pl
from jax.experimental.pallas import tpu as pltpu
from jax.experimental.pallas import tpu_sc as plsc


def _sc_gather(table, idx):
    V, D = table.shape
    B = idx.shape[0]
    info = plsc.get_sparse_core_info()
    NC, NS = info.num_cores, info.num_subcores
    NW = NC * NS
    assert B % NW == 0
    b_per_w = B // NW
    mesh = plsc.VectorSubcoreMesh(core_axis_name="c", subcore_axis_name="s")

    @functools.partial(
        pl.kernel,
        mesh=mesh,
        out_type=jax.ShapeDtypeStruct((B, D), jnp.float32),
        scratch_types=[
            pltpu.VMEM((b_per_w,), jnp.int32),
            pltpu.VMEM((b_per_w, D), jnp.float32),
            pltpu.SemaphoreType.DMA,
        ],
        compiler_params=pltpu.CompilerParams(use_tc_tiling_on_sc=False),
    )
    def k(table_hbm, idx_hbm, out_hbm, idx_v, rows_v, sem):
        wid = lax.axis_index("s") * NC + lax.axis_index("c")
        base = wid * b_per_w
        pltpu.sync_copy(idx_hbm.at[pl.ds(base, b_per_w)], idx_v)
        pltpu.async_copy(table_hbm.at[idx_v], rows_v, sem).wait()
        pltpu.sync_copy(rows_v, out_hbm.at[pl.ds(base, b_per_w)])

    return k(table, idx)


def _proj_body(flat_ref, w_ref, b_ref, out_ref):
    out_ref[...] = (
        lax.dot_general(
            flat_ref[...].astype(jnp.bfloat16),
            w_ref[...].astype(jnp.bfloat16),
            dimension_numbers=(((1,), (1,)), ((), ())),
            preferred_element_type=jnp.float32,
        )
        + b_ref[...]
    )


def _projection(flat, W, b2d, vblk):
    B, K = flat.shape
    V = W.shape[0]
    nblk = (V + vblk - 1) // vblk
    return pl.pallas_call(
        _proj_body,
        grid=(nblk,),
        in_specs=[
            pl.BlockSpec((B, K), lambda j: (0, 0)),
            pl.BlockSpec((vblk, K), lambda j: (j, 0)),
            pl.BlockSpec((1, vblk), lambda j: (0, j)),
        ],
        out_specs=pl.BlockSpec((B, vblk), lambda j: (0, j)),
        out_shape=jax.ShapeDtypeStruct((B, V), jnp.float32),
    )(flat, W, b2d)


def kernel(inputs, emb_table, W, b):
    api_seq = inputs[0]
    B, N = api_seq.shape
    D = emb_table.shape[1]
    idx = api_seq.reshape(B * N)
    rows = _sc_gather(emb_table, idx)
    flat = rows.reshape(B, N * D)
    return _projection(flat, W, b.reshape(1, -1), vblk=4096)
